# Initial kernel scaffold; baseline (speedup 1.0000x reference)
import jax
import jax.numpy as jnp
from jax.experimental import pallas as pl


def kernel(h, edge_attr, Wp, bp, g, mu, sigma, gb, fc1_w, fc1_b, fc2_w, fc2_b, edge_index, batch):
    raise NotImplementedError("write your pallas kernel here")


# speedup vs baseline: 1.0127x; 1.0127x over previous
import functools
import jax
import jax.numpy as jnp
from jax.experimental import pallas as pl
from jax.experimental.pallas import tpu as pltpu

N_NODES = 10000
N_EDGES = 320000
D = 128
K = 3
OUT = 10
NUM_GRAPHS = 64
EPS = 1e-15


def _xg_body(h_ref, g_ref, out_ref):
    out_ref[...] = jnp.dot(h_ref[...], g_ref[...],
                           preferred_element_type=jnp.float32)


def _xg(h, g_l):
    return pl.pallas_call(
        _xg_body,
        out_shape=jax.ShapeDtypeStruct((N_NODES, K * D), jnp.float32),
    )(h, g_l)


def _w_body(p_ref, ea0_ref, ea1_ref, w0_ref, w1_ref, w2_ref):
    ea0 = ea0_ref[...]
    ea1 = ea1_ref[...]
    p0 = jnp.tanh(ea0 * p_ref[0] + ea1 * p_ref[2] + p_ref[4])
    p1 = jnp.tanh(ea0 * p_ref[1] + ea1 * p_ref[3] + p_ref[5])
    outs = (w0_ref, w1_ref, w2_ref)
    for k in range(K):
        m0 = p_ref[6 + 2 * k]
        m1 = p_ref[7 + 2 * k]
        i0 = p_ref[12 + 2 * k]
        i1 = p_ref[13 + 2 * k]
        gauss = (p0 - m0) ** 2 * i0 + (p1 - m1) ** 2 * i1
        outs[k][...] = jnp.exp(-0.5 * gauss)


def _edge_w(ea0, ea1, Wp_l, bp_l, mu_l, sigma_l):
    inv = 1.0 / (EPS + sigma_l ** 2)
    pvec = jnp.concatenate([
        Wp_l.reshape(-1), bp_l.reshape(-1), mu_l.reshape(-1), inv.reshape(-1)
    ]).astype(jnp.float32)
    rows = N_EDGES // 128
    shp = jax.ShapeDtypeStruct((rows, 128), jnp.float32)
    return pl.pallas_call(
        _w_body,
        in_specs=[
            pl.BlockSpec(memory_space=pltpu.SMEM),
            pl.BlockSpec(memory_space=pltpu.VMEM),
            pl.BlockSpec(memory_space=pltpu.VMEM),
        ],
        out_shape=(shp, shp, shp),
    )(pvec, ea0, ea1)


def _head_body(h_ref, b_ref, fc1w_ref, fc1b_ref, fc2w_ref, fc2b_ref,
               out_ref):
    h = h_ref[...]
    bcol = b_ref[...]
    gids = jax.lax.broadcasted_iota(jnp.int32, (N_NODES, NUM_GRAPHS), 1)
    onehot = (bcol == gids).astype(jnp.float32)
    sums = jnp.dot(onehot.T, h, preferred_element_type=jnp.float32)
    counts = jnp.sum(onehot, axis=0)
    hg = sums / jnp.clip(counts, 1.0)[:, None]
    hg = jnp.dot(hg, fc1w_ref[...], preferred_element_type=jnp.float32)
    hg = hg + fc1b_ref[...][None, :]
    hg = jnp.where(hg > 0, hg, jnp.exp(jnp.minimum(hg, 0.0)) - 1.0)
    hg = jnp.dot(hg, fc2w_ref[...], preferred_element_type=jnp.float32)
    hg = hg + fc2b_ref[...][None, :]
    m = jnp.max(hg, axis=0, keepdims=True)
    z = hg - m
    lse = jnp.log(jnp.sum(jnp.exp(z), axis=0, keepdims=True))
    out_ref[...] = z - lse


def _head(h, batch, fc1_w, fc1_b, fc2_w, fc2_b):
    return pl.pallas_call(
        _head_body,
        out_shape=jax.ShapeDtypeStruct((NUM_GRAPHS, OUT), jnp.float32),
    )(h, batch.reshape(N_NODES, 1), fc1_w, fc1_b, fc2_w, fc2_b)


def kernel(h, edge_attr, Wp, bp, g, mu, sigma, gb, fc1_w, fc1_b, fc2_w,
           fc2_b, edge_index, batch):
    src = edge_index[0]
    dst = edge_index[1]
    ea0 = edge_attr[:, 0].reshape(N_EDGES // 128, 128)
    ea1 = edge_attr[:, 1].reshape(N_EDGES // 128, 128)

    for l in range(4):
        xg = _xg(h, g[l])
        w0, w1, w2 = _edge_w(ea0, ea1, Wp[l], bp[l], mu[l], sigma[l])
        w = jnp.stack([w0.reshape(-1), w1.reshape(-1), w2.reshape(-1)],
                      axis=1)
        xj = xg[src].reshape(-1, K, D)
        msg = (xj * w[:, :, None]).sum(axis=1)
        agg = jax.ops.segment_max(msg, dst, num_segments=N_NODES)
        agg = jnp.where(jnp.isfinite(agg), agg, 0.0)
        h = jax.nn.relu(agg + gb[l][None, :])

    return _head(h, batch, fc1_w, fc1_b, fc2_w, fc2_b)
